# baseline (speedup 1.0000x reference)
import functools

import jax
import jax.numpy as jnp
import numpy as np
from jax import lax
from jax.experimental import pallas as pl
from jax.experimental.pallas import tpu as pltpu
from jax.experimental.pallas import tpu_sc as plsc

_VOCAB = 100000
_D = 64
_B = 1024
_C = 20
_NCTX = 8
_LT = 3
_LP = 8
_LC = 5
_LQ = 50
_INF = 1e20

_NZPAD = 1024
_ROWW = 128

_TPC = _LT + _LP + _NCTX * _LC
_CHUNK = 128
_CPU_ = 5
_NUNIT = _C // _CPU_
_USLOT = 2 * _CHUNK
_NPAD = _CHUNK + _NUNIT * _USLOT
_NCHUNK = _NPAD // _CHUNK
_NLEN = 224

_OFF_QL = 0
_OFF_NC = 1
_OFF_TL = 2
_OFF_PL = 22
_OFF_NUM = 42
_OFF_CL = 62
_OFF_PAD = 222

_NW = 32
_BPW = _B // _NW

_UCHUNKS = [(0, 1, 2), (3, 4), (5, 6), (7, 8)]
_UBUF = [0, 1, 0, 1]
_UBASE = [_CHUNK, -255, -510, -765]


def _build_consts():
    pos = np.ones((_NPAD,), np.int32)
    off = np.full((_NPAD,), _OFF_PAD, np.int32)
    pos[0:_LQ] = np.arange(_LQ)
    off[0:_LQ] = _OFF_QL
    for c in range(_C):
        u, cu = divmod(c, _CPU_)
        b = _CHUNK + _USLOT * u + _TPC * cu
        pos[b:b + _LT] = np.arange(_LT)
        off[b:b + _LT] = _OFF_TL + c
        pos[b + _LT:b + _LT + _LP] = np.arange(_LP)
        off[b + _LT:b + _LT + _LP] = _OFF_PL + c
        for n in range(_NCTX):
            bb = b + _LT + _LP + _LC * n
            pos[bb:bb + _LC] = np.arange(_LC)
            off[bb:bb + _LC] = _OFF_CL + _NCTX * c + n
    spread = (_VOCAB + (np.arange(_NPAD, dtype=np.int64) * 89) % _NZPAD
              ).astype(np.int32)
    shp = (_NCHUNK, _CHUNK)
    return pos.reshape(shp), off.reshape(shp), spread.reshape(shp)


_POS_NP, _OFF_NP, _SPR_NP = _build_consts()
_GBASE_NP = np.array(
    [_OFF_TL, _OFF_PL, _OFF_NUM] + [_OFF_CL + n for n in range(_NCTX)]
    + [_OFF_PAD] * 5, np.int32)
_GMULT_NP = np.array([1, 1, 1] + [_NCTX] * _NCTX + [0] * 5, np.int32)
_AUX_NP = np.stack([np.arange(16, dtype=np.int32), _GBASE_NP, _GMULT_NP,
                    np.zeros(16, np.int32)])
_RLUT_NP = (1.0 / np.maximum(np.arange(64), 1)).astype(np.float32)


def _b16(x, i):
    return x.at[jnp.full((16,), i, jnp.int32)].get(mode="promise_in_bounds")


def _bownet_sc_body(w_hbm, idx_hbm, lens_hbm, pos_hbm, off_hbm, spr_hbm,
                    aux_hbm, rlut_hbm, out_hbm, pos_v, off_v, spr_v, idx_a,
                    idx_b, lens_a, lens_b, rows_a, rows_b, score_v, aux_v,
                    rlut_v, sem_a, sem_b, sem_ia, sem_ib):
    wid = lax.axis_index("s") * 2 + lax.axis_index("c")
    pltpu.sync_copy(pos_hbm, pos_v)
    pltpu.sync_copy(off_hbm, off_v)
    pltpu.sync_copy(spr_hbm, spr_v)
    pltpu.sync_copy(aux_hbm, aux_v)
    pltpu.sync_copy(rlut_hbm, rlut_v)
    bufs = (rows_a, rows_b)
    sems = (sem_a, sem_b)
    idxs = (idx_a, idx_b)
    lenss = (lens_a, lens_b)
    isems = (sem_ia, sem_ib)

    def _fire(u, idx_v):
        buf, sem = bufs[_UBUF[u]], sems[_UBUF[u]]
        for i, j in enumerate(_UCHUNKS[u]):
            pltpu.async_copy(w_hbm.at[idx_v.at[j]],
                             buf.at[pl.ds(i * _CHUNK, _CHUNK)], sem)

    def _drain(u):
        buf, sem = bufs[_UBUF[u]], sems[_UBUF[u]]
        for _ in _UCHUNKS[u]:
            pltpu.make_async_copy(
                w_hbm.at[idx_a.at[0]], buf.at[pl.ds(0, _CHUNK)], sem).wait()

    def _stage(b, p):
        pltpu.async_copy(idx_hbm.at[b], idxs[p], isems[p])
        pltpu.async_copy(lens_hbm.at[b], lenss[p], isems[p])

    def _stage_wait(p):
        pltpu.make_async_copy(idx_hbm.at[0], idxs[p], isems[p]).wait()
        pltpu.make_async_copy(lens_hbm.at[0], lenss[p], isems[p]).wait()

    _stage(wid * _BPW, 0)

    @pl.loop(0, _BPW // 2)
    def _pair(i):
      for p in (0, 1):
        bl = 2 * i + p
        b = wid * _BPW + bl
        idx_v = idxs[p]
        lens_v = lenss[p]
        _stage_wait(p)

        @pl.when(bl + 1 < _BPW)
        def _pref():
            _stage(b + 1, 1 - p)

        @pl.loop(0, _NCHUNK)
        def _mask(j):
            for k in range(_CHUNK // 16):
                sl = pl.ds(k * 16, 16)
                raw = idx_v[j, sl]
                ln = plsc.load_gather(lens_v, [off_v[j, sl]])
                idx_v[j, sl] = jnp.where(pos_v[j, sl] < ln, raw, spr_v[j, sl])

        lv0 = lens_v[pl.ds(0, 16)]
        rlv0 = plsc.load_gather(rlut_v, [lv0])
        rqlv = _b16(rlv0, _OFF_QL)
        nc = lv0[_OFF_NC]

        iota = aux_v[0, :]
        gbase = aux_v[1, :]
        gmult = aux_v[2, :]

        _fire(0, idx_v)

        @pl.when(nc > _CPU_)
        def _f1():
            _fire(1, idx_v)

        _drain(0)

        qs = [rows_a[0, pl.ds(k * 16, 16)] for k in range(4)]
        for t in range(1, _LQ):
            for k in range(4):
                qs[k] = qs[k] + rows_a[t, pl.ds(k * 16, 16)]
        q = tuple(a * rqlv for a in qs)

        def _cand(u):
            buf = bufs[_UBUF[u]]

            def body(c, carry):
                s0, s1, cvec = carry
                g = plsc.load_gather(lens_v, [gbase + cvec * gmult])
                r = plsc.load_gather(rlut_v, [g])
                rtv = _b16(r, 0)
                rpv = _b16(r, 1)
                rnv = _b16(r, 2)
                numv = _b16(g, 2)
                wv = jnp.where((iota >= 3) & (iota - 3 < numv), rnv * r, 0.0)
                bt = _UBASE[u] + _TPC * c
                bp = bt + _LT
                bc = bp + _LP
                key = []
                for k in range(4):
                    sl = pl.ds(k * 16, 16)
                    ts = (buf[bt, sl] + buf[bt + 1, sl] + buf[bt + 2, sl])
                    ps = buf[bp, sl]
                    for j in range(1, _LP):
                        ps = ps + buf[bp + j, sl]
                    key.append(ts * rtv + ps * rpv)
                for n in range(_NCTX):
                    wnv = _b16(wv, 3 + n)
                    base = bc + _LC * n
                    for k in range(4):
                        sl = pl.ds(k * 16, 16)
                        s = buf[base, sl]
                        for j in range(1, _LC):
                            s = s + buf[base + j, sl]
                        key[k] = key[k] + wnv * s
                v = (key[0] * q[0] + key[1] * q[1] + key[2] * q[2]
                     + key[3] * q[3])
                scv = _b16(plsc.cumsum(v), 15)
                s0 = jnp.where(iota == cvec, scv, s0)
                s1 = jnp.where(iota == cvec - 16, scv, s1)
                return s0, s1, cvec + 1
            return body

        neg = jnp.full((16,), -_INF, jnp.float32)
        cz = jnp.full((16,), 0, jnp.int32)
        s0, s1, _ = lax.fori_loop(0, jnp.minimum(nc, _CPU_), _cand(0),
                                  (neg, neg, cz))
        score_v[bl, pl.ds(0, 16)] = s0
        score_v[bl, pl.ds(16, 16)] = s1

        def _unit_pass(u):
            lo = _CPU_ * u

            @pl.when(nc > lo)
            def _go():
                if u + 1 < _NUNIT:
                    @pl.when(nc > lo + _CPU_)
                    def _fn():
                        _fire(u + 1, idx_v)
                _drain(u)
                t0 = score_v[bl, pl.ds(0, 16)]
                t1 = score_v[bl, pl.ds(16, 16)]
                hi = jnp.minimum(nc, lo + _CPU_)
                cl0 = jnp.full((16,), lo, jnp.int32)
                u0, u1, _ = lax.fori_loop(lo, hi, _cand(u), (t0, t1, cl0))
                score_v[bl, pl.ds(0, 16)] = u0
                score_v[bl, pl.ds(16, 16)] = u1

        for u in range(1, _NUNIT):
            _unit_pass(u)

    pltpu.sync_copy(score_v, out_hbm.at[pl.ds(wid * _BPW, _BPW)])


@functools.lru_cache(maxsize=None)
def _get_sc_call():
    mesh = plsc.VectorSubcoreMesh(
        core_axis_name="c", subcore_axis_name="s",
        num_cores=2, num_subcores=16)
    return pl.kernel(
        _bownet_sc_body,
        out_type=jax.ShapeDtypeStruct((_B, 32), jnp.float32),
        mesh=mesh,
        compiler_params=pltpu.CompilerParams(needs_layout_passes=False),
        scratch_types=[
            pltpu.VMEM((_NCHUNK, _CHUNK), jnp.int32),
            pltpu.VMEM((_NCHUNK, _CHUNK), jnp.int32),
            pltpu.VMEM((_NCHUNK, _CHUNK), jnp.int32),
            pltpu.VMEM((_NCHUNK, _CHUNK), jnp.int32),
            pltpu.VMEM((_NCHUNK, _CHUNK), jnp.int32),
            pltpu.VMEM((_NLEN,), jnp.int32),
            pltpu.VMEM((_NLEN,), jnp.int32),
            pltpu.VMEM((3 * _CHUNK, _ROWW), jnp.float32),
            pltpu.VMEM((2 * _CHUNK, _ROWW), jnp.float32),
            pltpu.VMEM((_BPW, 32), jnp.float32),
            pltpu.VMEM((4, 16), jnp.int32),
            pltpu.VMEM((64,), jnp.float32),
            pltpu.SemaphoreType.DMA,
            pltpu.SemaphoreType.DMA,
            pltpu.SemaphoreType.DMA,
            pltpu.SemaphoreType.DMA,
        ],
    )


def kernel(W, queries, query_lengths, num_cands, x_type_bow, x_type_bow_len,
           x_path_bow, x_path_bow_len, x_ctx_ent, x_ctx_ent_len,
           x_ctx_ent_num):
    i32 = jnp.int32
    Wp = jnp.pad(W, ((0, _NZPAD), (0, _ROWW - _D)))
    cand = jnp.concatenate([
        x_type_bow,
        x_path_bow,
        x_ctx_ent.reshape(_B, _C, _NCTX * _LC),
    ], axis=2)
    units = jnp.pad(cand.reshape(_B, _NUNIT, _CPU_ * _TPC),
                    ((0, 0), (0, 0), (0, _USLOT - _CPU_ * _TPC)))
    idx = jnp.concatenate([
        jnp.pad(queries, ((0, 0), (0, _CHUNK - _LQ))),
        units.reshape(_B, _NUNIT * _USLOT),
    ], axis=1).reshape(_B, _NCHUNK, _CHUNK)
    lens = jnp.concatenate([
        query_lengths[:, None],
        num_cands[:, None],
        x_type_bow_len,
        x_path_bow_len,
        x_ctx_ent_num,
        x_ctx_ent_len.reshape(_B, _C * _NCTX),
        jnp.ones((_B, 2), i32),
    ], axis=1)
    out = _get_sc_call()(Wp, idx, lens, jnp.asarray(_POS_NP),
                         jnp.asarray(_OFF_NP), jnp.asarray(_SPR_NP),
                         jnp.asarray(_AUX_NP), jnp.asarray(_RLUT_NP))
    return out[:, :_C]


# speedup vs baseline: 1.2164x; 1.2164x over previous
import functools

import jax
import jax.numpy as jnp
import numpy as np
from jax import lax
from jax.experimental import pallas as pl
from jax.experimental.pallas import tpu as pltpu
from jax.experimental.pallas import tpu_sc as plsc

_VOCAB = 100000
_D = 64
_B = 1024
_C = 20
_NCTX = 8
_LT = 3
_LP = 8
_LC = 5
_LQ = 50
_INF = 1e20

_NZPAD = 1024
_ROWW = 128

_TPC = _LT + _LP + _NCTX * _LC
_CHUNK = 128
_CPU_ = 5
_NUNIT = _C // _CPU_
_USLOT = 2 * _CHUNK
_NPAD = _CHUNK + _NUNIT * _USLOT
_NCHUNK = _NPAD // _CHUNK
_NLEN = 224

_OFF_QL = 0
_OFF_NC = 1
_OFF_TL = 2
_OFF_PL = 22
_OFF_NUM = 42
_OFF_CL = 62
_OFF_PAD = 222

_NW = 32
_BPW = _B // _NW

_UCHUNKS = [(0, 1, 2), (3, 4), (5, 6), (7, 8)]
_UBUF = [0, 1, 0, 1]
_UBASE = [_CHUNK, -255, -510, -765]


def _build_consts():
    pos = np.ones((_NPAD,), np.int32)
    off = np.full((_NPAD,), _OFF_PAD, np.int32)
    pos[0:_LQ] = np.arange(_LQ)
    off[0:_LQ] = _OFF_QL
    for c in range(_C):
        u, cu = divmod(c, _CPU_)
        b = _CHUNK + _USLOT * u + _TPC * cu
        pos[b:b + _LT] = np.arange(_LT)
        off[b:b + _LT] = _OFF_TL + c
        pos[b + _LT:b + _LT + _LP] = np.arange(_LP)
        off[b + _LT:b + _LT + _LP] = _OFF_PL + c
        for n in range(_NCTX):
            bb = b + _LT + _LP + _LC * n
            pos[bb:bb + _LC] = np.arange(_LC)
            off[bb:bb + _LC] = _OFF_CL + _NCTX * c + n
    spread = (_VOCAB + (np.arange(_NPAD, dtype=np.int64) * 89) % _NZPAD
              ).astype(np.int32)
    shp = (_NCHUNK, _CHUNK)
    return pos.reshape(shp), off.reshape(shp), spread.reshape(shp)


_POS_NP, _OFF_NP, _SPR_NP = _build_consts()
_GBASE_NP = np.array(
    [_OFF_TL, _OFF_PL, _OFF_NUM] + [_OFF_CL + n for n in range(_NCTX)]
    + [_OFF_PAD] * 5, np.int32)
_GMULT_NP = np.array([1, 1, 1] + [_NCTX] * _NCTX + [0] * 5, np.int32)
_AUX_NP = np.stack([np.arange(16, dtype=np.int32), _GBASE_NP, _GMULT_NP,
                    np.zeros(16, np.int32)])
_RLUT_NP = (1.0 / np.maximum(np.arange(64), 1)).astype(np.float32)


def _b16(x, i):
    return x.at[jnp.full((16,), i, jnp.int32)].get(mode="promise_in_bounds")


def _bownet_sc_body(w_hbm, idx_hbm, lens_hbm, pos_hbm, off_hbm, spr_hbm,
                    aux_hbm, rlut_hbm, out_hbm, pos_v, off_v, spr_v, idx_a,
                    idx_b, lens_a, lens_b, rows_a, rows_b, score_v, aux_v,
                    rlut_v, sem_a, sem_b, sem_ia, sem_ib):
    wid = lax.axis_index("s") * 2 + lax.axis_index("c")
    pltpu.sync_copy(pos_hbm, pos_v)
    pltpu.sync_copy(off_hbm, off_v)
    pltpu.sync_copy(spr_hbm, spr_v)
    pltpu.sync_copy(aux_hbm, aux_v)
    pltpu.sync_copy(rlut_hbm, rlut_v)
    bufs = (rows_a, rows_b)
    sems = (sem_a, sem_b)
    idxs = (idx_a, idx_b)
    lenss = (lens_a, lens_b)
    isems = (sem_ia, sem_ib)

    def _fire(u, idx_v):
        buf, sem = bufs[_UBUF[u]], sems[_UBUF[u]]
        for i, j in enumerate(_UCHUNKS[u]):
            pltpu.async_copy(w_hbm.at[idx_v.at[j]],
                             buf.at[pl.ds(i * _CHUNK, _CHUNK)], sem)

    def _drain(u):
        buf, sem = bufs[_UBUF[u]], sems[_UBUF[u]]
        for _ in _UCHUNKS[u]:
            pltpu.make_async_copy(
                w_hbm.at[idx_a.at[0]], buf.at[pl.ds(0, _CHUNK)], sem).wait()

    def _stage(b, p):
        pltpu.async_copy(idx_hbm.at[b], idxs[p], isems[p])
        pltpu.async_copy(lens_hbm.at[b], lenss[p], isems[p])

    def _stage_wait(p):
        pltpu.make_async_copy(idx_hbm.at[0], idxs[p], isems[p]).wait()
        pltpu.make_async_copy(lens_hbm.at[0], lenss[p], isems[p]).wait()

    _stage(wid * _BPW, 0)

    @pl.loop(0, _BPW // 2)
    def _pair(i):
      for p in (0, 1):
        bl = 2 * i + p
        b = wid * _BPW + bl
        idx_v = idxs[p]
        lens_v = lenss[p]
        _stage_wait(p)

        @pl.when(bl + 1 < _BPW)
        def _pref():
            _stage(b + 1, 1 - p)

        @pl.loop(0, _NCHUNK)
        def _mask(j):
            for k in range(_CHUNK // 16):
                sl = pl.ds(k * 16, 16)
                raw = idx_v[j, sl]
                ln = plsc.load_gather(lens_v, [off_v[j, sl]])
                idx_v[j, sl] = jnp.where(pos_v[j, sl] < ln, raw, spr_v[j, sl])

        lv0 = lens_v[pl.ds(0, 16)]
        rlv0 = plsc.load_gather(rlut_v, [lv0])
        rqlv = _b16(rlv0, _OFF_QL)
        nc = lv0[_OFF_NC]

        iota = aux_v[0, :]
        gbase = aux_v[1, :]
        gmult = aux_v[2, :]

        _fire(0, idx_v)

        @pl.when(nc > _CPU_)
        def _f1():
            _fire(1, idx_v)

        _drain(0)

        def _qacc(t, acc):
            return tuple(acc[k] + rows_a[t, pl.ds(k * 16, 16)]
                         for k in range(4))
        z = jnp.zeros((16,), jnp.float32)
        qs = lax.fori_loop(0, _LQ, _qacc, (z, z, z, z))
        q = tuple(a * rqlv for a in qs)

        def _cand(u):
            buf = bufs[_UBUF[u]]

            def body(c, carry):
                s0, s1, cvec = carry
                g = plsc.load_gather(lens_v, [gbase + cvec * gmult])
                r = plsc.load_gather(rlut_v, [g])
                rtv = _b16(r, 0)
                rpv = _b16(r, 1)
                rnv = _b16(r, 2)
                numv = _b16(g, 2)
                wv = jnp.where((iota >= 3) & (iota - 3 < numv), rnv * r, 0.0)
                bt = _UBASE[u] + _TPC * c
                bp = bt + _LT
                bc = bp + _LP
                key = []
                for k in range(4):
                    sl = pl.ds(k * 16, 16)
                    ts = (buf[bt, sl] + buf[bt + 1, sl] + buf[bt + 2, sl])
                    ps = buf[bp, sl]
                    for j in range(1, _LP):
                        ps = ps + buf[bp + j, sl]
                    key.append(ts * rtv + ps * rpv)
                def _ctx(n, kc):
                    wnv = wv.at[jnp.full((16,), 3, jnp.int32) + n].get(
                        mode="promise_in_bounds")
                    base = bc + _LC * n
                    out = []
                    for k in range(4):
                        sl = pl.ds(k * 16, 16)
                        s = buf[base, sl]
                        for j in range(1, _LC):
                            s = s + buf[base + j, sl]
                        out.append(kc[k] + wnv * s)
                    return tuple(out)
                key = list(lax.fori_loop(0, _NCTX, _ctx, tuple(key)))
                v = (key[0] * q[0] + key[1] * q[1] + key[2] * q[2]
                     + key[3] * q[3])
                scv = _b16(plsc.cumsum(v), 15)
                s0 = jnp.where(iota == cvec, scv, s0)
                s1 = jnp.where(iota == cvec - 16, scv, s1)
                return s0, s1, cvec + 1
            return body

        neg = jnp.full((16,), -_INF, jnp.float32)
        cz = jnp.full((16,), 0, jnp.int32)
        s0, s1, _ = lax.fori_loop(0, jnp.minimum(nc, _CPU_), _cand(0),
                                  (neg, neg, cz))
        score_v[bl, pl.ds(0, 16)] = s0
        score_v[bl, pl.ds(16, 16)] = s1

        def _unit_pass(u):
            lo = _CPU_ * u

            @pl.when(nc > lo)
            def _go():
                if u + 1 < _NUNIT:
                    @pl.when(nc > lo + _CPU_)
                    def _fn():
                        _fire(u + 1, idx_v)
                _drain(u)
                t0 = score_v[bl, pl.ds(0, 16)]
                t1 = score_v[bl, pl.ds(16, 16)]
                hi = jnp.minimum(nc, lo + _CPU_)
                cl0 = jnp.full((16,), lo, jnp.int32)
                u0, u1, _ = lax.fori_loop(lo, hi, _cand(u), (t0, t1, cl0))
                score_v[bl, pl.ds(0, 16)] = u0
                score_v[bl, pl.ds(16, 16)] = u1

        for u in range(1, _NUNIT):
            _unit_pass(u)

    pltpu.sync_copy(score_v, out_hbm.at[pl.ds(wid * _BPW, _BPW)])


@functools.lru_cache(maxsize=None)
def _get_sc_call():
    mesh = plsc.VectorSubcoreMesh(
        core_axis_name="c", subcore_axis_name="s",
        num_cores=2, num_subcores=16)
    return pl.kernel(
        _bownet_sc_body,
        out_type=jax.ShapeDtypeStruct((_B, 32), jnp.float32),
        mesh=mesh,
        compiler_params=pltpu.CompilerParams(needs_layout_passes=False),
        scratch_types=[
            pltpu.VMEM((_NCHUNK, _CHUNK), jnp.int32),
            pltpu.VMEM((_NCHUNK, _CHUNK), jnp.int32),
            pltpu.VMEM((_NCHUNK, _CHUNK), jnp.int32),
            pltpu.VMEM((_NCHUNK, _CHUNK), jnp.int32),
            pltpu.VMEM((_NCHUNK, _CHUNK), jnp.int32),
            pltpu.VMEM((_NLEN,), jnp.int32),
            pltpu.VMEM((_NLEN,), jnp.int32),
            pltpu.VMEM((3 * _CHUNK, _ROWW), jnp.float32),
            pltpu.VMEM((2 * _CHUNK, _ROWW), jnp.float32),
            pltpu.VMEM((_BPW, 32), jnp.float32),
            pltpu.VMEM((4, 16), jnp.int32),
            pltpu.VMEM((64,), jnp.float32),
            pltpu.SemaphoreType.DMA,
            pltpu.SemaphoreType.DMA,
            pltpu.SemaphoreType.DMA,
            pltpu.SemaphoreType.DMA,
        ],
    )


def kernel(W, queries, query_lengths, num_cands, x_type_bow, x_type_bow_len,
           x_path_bow, x_path_bow_len, x_ctx_ent, x_ctx_ent_len,
           x_ctx_ent_num):
    i32 = jnp.int32
    Wp = jnp.pad(W, ((0, _NZPAD), (0, _ROWW - _D)))
    cand = jnp.concatenate([
        x_type_bow,
        x_path_bow,
        x_ctx_ent.reshape(_B, _C, _NCTX * _LC),
    ], axis=2)
    units = jnp.pad(cand.reshape(_B, _NUNIT, _CPU_ * _TPC),
                    ((0, 0), (0, 0), (0, _USLOT - _CPU_ * _TPC)))
    idx = jnp.concatenate([
        jnp.pad(queries, ((0, 0), (0, _CHUNK - _LQ))),
        units.reshape(_B, _NUNIT * _USLOT),
    ], axis=1).reshape(_B, _NCHUNK, _CHUNK)
    lens = jnp.concatenate([
        query_lengths[:, None],
        num_cands[:, None],
        x_type_bow_len,
        x_path_bow_len,
        x_ctx_ent_num,
        x_ctx_ent_len.reshape(_B, _C * _NCTX),
        jnp.ones((_B, 2), i32),
    ], axis=1)
    out = _get_sc_call()(Wp, idx, lens, jnp.asarray(_POS_NP),
                         jnp.asarray(_OFF_NP), jnp.asarray(_SPR_NP),
                         jnp.asarray(_AUX_NP), jnp.asarray(_RLUT_NP))
    return out[:, :_C]
